# baseline (speedup 1.0000x reference)
import functools

import jax
import jax.numpy as jnp
from jax import lax
from jax.experimental import pallas as pl
from jax.experimental.pallas import tpu as pltpu
from jax.experimental.pallas import tpu_sc as plsc

VOCAB = 21
EMB_DIM = 128
BATCH = 4096
SEQ_LEN = 50
_LPAD = SEQ_LEN + 2

_info = plsc.get_sparse_core_info()
_NC = _info.num_cores
_NS = _info.num_subcores
_NW = _NC * _NS
_BCHUNK = BATCH // _NW
_LANES = 16


def _emb_body(idx_hbm, table_hbm, out_hbm,
              patt_v, idxT_v, rows0_v, rows1_v,
              sem_i, sem_r0, sem_r1, sem_s0, sem_s1):
    wid = lax.axis_index("s") * _NC + lax.axis_index("c")
    b0 = wid * _BCHUNK
    lane = lax.iota(jnp.int32, _LANES)

    def build(l, carry):
        l_eff = jnp.minimum(l, SEQ_LEN - 1)
        for k in range(_BCHUNK // _LANES):
            patt_v[l, pl.ds(k * _LANES, _LANES)] = (
                (b0 + k * _LANES + lane) * SEQ_LEN + l_eff)
        return carry
    lax.fori_loop(0, _LPAD, build, 0)

    def fire(l, carry):
        pltpu.async_copy(idx_hbm.at[patt_v.at[l]], idxT_v.at[l], sem_i)
        return carry
    lax.fori_loop(0, _LPAD, fire, 0)

    def drain(l, carry):
        pltpu.make_async_copy(idx_hbm.at[patt_v.at[l]], idxT_v.at[l],
                              sem_i).wait()
        return carry
    lax.fori_loop(0, _LPAD, drain, 0)

    def gather(l, rows, sem):
        pltpu.async_copy(table_hbm.at[idxT_v.at[l]], rows, sem)

    def gather_wait(l, rows, sem):
        pltpu.make_async_copy(table_hbm.at[idxT_v.at[l]], rows, sem).wait()

    def scatter(l, rows, sem):
        pltpu.async_copy(rows, out_hbm.at[l, pl.ds(b0, _BCHUNK), :], sem)

    def scatter_wait(l, rows, sem):
        pltpu.make_async_copy(rows, out_hbm.at[l, pl.ds(b0, _BCHUNK), :],
                              sem).wait()

    gather(0, rows0_v, sem_r0)
    gather(1, rows1_v, sem_r1)

    def pair(p, carry):
        l0 = 2 * p
        gather_wait(l0, rows0_v, sem_r0)
        scatter(l0, rows0_v, sem_s0)
        gather_wait(l0 + 1, rows1_v, sem_r1)
        scatter(l0 + 1, rows1_v, sem_s1)
        scatter_wait(l0, rows0_v, sem_s0)
        gather(l0 + 2, rows0_v, sem_r0)
        scatter_wait(l0 + 1, rows1_v, sem_s1)
        gather(l0 + 3, rows1_v, sem_r1)
        return carry
    lax.fori_loop(0, SEQ_LEN // 2, pair, 0)

    gather_wait(SEQ_LEN, rows0_v, sem_r0)
    gather_wait(SEQ_LEN + 1, rows1_v, sem_r1)


_emb_kernel = functools.partial(
    pl.kernel,
    mesh=plsc.VectorSubcoreMesh(core_axis_name="c", subcore_axis_name="s"),
    out_type=jax.ShapeDtypeStruct((SEQ_LEN, BATCH, EMB_DIM), jnp.float32),
    scratch_types=[
        pltpu.VMEM((_LPAD, _BCHUNK), jnp.int32),
        pltpu.VMEM((_LPAD, _BCHUNK), jnp.int32),
        pltpu.VMEM((_BCHUNK, EMB_DIM), jnp.float32),
        pltpu.VMEM((_BCHUNK, EMB_DIM), jnp.float32),
        pltpu.SemaphoreType.DMA,
        pltpu.SemaphoreType.DMA,
        pltpu.SemaphoreType.DMA,
        pltpu.SemaphoreType.DMA,
        pltpu.SemaphoreType.DMA,
    ],
)(_emb_body)


def kernel(indices, table):
    return _emb_kernel(indices.astype(jnp.int32).reshape(-1), table)


# speedup vs baseline: 1.0340x; 1.0340x over previous
import functools

import jax
import jax.numpy as jnp
from jax import lax
from jax.experimental import pallas as pl
from jax.experimental.pallas import tpu as pltpu
from jax.experimental.pallas import tpu_sc as plsc

VOCAB = 21
EMB_DIM = 128
BATCH = 4096
SEQ_LEN = 50
_ROWS = BATCH * SEQ_LEN

_info = plsc.get_sparse_core_info()
_NC = _info.num_cores
_NS = _info.num_subcores
_NW = _NC * _NS
_WROWS = _ROWS // _NW
_CHUNK = 128
_NCHUNK = _WROWS // _CHUNK


def _tr_body(idx_ref, out_ref):
    out_ref[...] = jnp.transpose(idx_ref[...], (1, 0))


_transpose = pl.pallas_call(
    _tr_body,
    out_shape=jax.ShapeDtypeStruct((SEQ_LEN, BATCH), jnp.int32),
)


def _emb_body(idxT_hbm, table_hbm, out_hbm,
              idxT_v, rows0_v, rows1_v,
              sem_i, sem_r0, sem_r1, sem_s0, sem_s1):
    wid = lax.axis_index("s") * _NC + lax.axis_index("c")
    r0 = wid * _WROWS

    pltpu.sync_copy(idxT_hbm.at[wid], idxT_v)

    def gather(c, rows, sem):
        pltpu.async_copy(table_hbm.at[idxT_v.at[c]], rows, sem)

    def gather_wait(c, rows, sem):
        pltpu.make_async_copy(table_hbm.at[idxT_v.at[c]], rows, sem).wait()

    def scatter(c, rows, sem):
        pltpu.async_copy(rows, out_hbm.at[pl.ds(r0 + c * _CHUNK, _CHUNK), :],
                         sem)

    def scatter_wait(c, rows, sem):
        pltpu.make_async_copy(rows,
                              out_hbm.at[pl.ds(r0 + c * _CHUNK, _CHUNK), :],
                              sem).wait()

    gather(0, rows0_v, sem_r0)
    gather(1, rows1_v, sem_r1)

    def pair(p, carry):
        c0 = 2 * p
        gather_wait(c0, rows0_v, sem_r0)
        scatter(c0, rows0_v, sem_s0)
        gather_wait(c0 + 1, rows1_v, sem_r1)
        scatter(c0 + 1, rows1_v, sem_s1)
        scatter_wait(c0, rows0_v, sem_s0)

        @pl.when(p < _NCHUNK // 2 - 1)
        def _():
            gather(c0 + 2, rows0_v, sem_r0)

        scatter_wait(c0 + 1, rows1_v, sem_s1)

        @pl.when(p < _NCHUNK // 2 - 1)
        def _():
            gather(c0 + 3, rows1_v, sem_r1)

        return carry

    lax.fori_loop(0, _NCHUNK // 2, pair, 0)


_emb_kernel = functools.partial(
    pl.kernel,
    mesh=plsc.VectorSubcoreMesh(core_axis_name="c", subcore_axis_name="s"),
    out_type=jax.ShapeDtypeStruct((_ROWS, EMB_DIM), jnp.float32),
    scratch_types=[
        pltpu.VMEM((_NCHUNK, _CHUNK), jnp.int32),
        pltpu.VMEM((_CHUNK, EMB_DIM), jnp.float32),
        pltpu.VMEM((_CHUNK, EMB_DIM), jnp.float32),
        pltpu.SemaphoreType.DMA,
        pltpu.SemaphoreType.DMA,
        pltpu.SemaphoreType.DMA,
        pltpu.SemaphoreType.DMA,
        pltpu.SemaphoreType.DMA,
    ],
)(_emb_body)


def kernel(indices, table):
    idx_t = _transpose(indices.astype(jnp.int32))
    idx_t = idx_t.reshape(_NW, _NCHUNK, _CHUNK)
    out = _emb_kernel(idx_t, table)
    return out.reshape(SEQ_LEN, BATCH, EMB_DIM)


# speedup vs baseline: 6.5316x; 6.3171x over previous
import functools

import jax
import jax.numpy as jnp
from jax import lax
from jax.experimental import pallas as pl
from jax.experimental.pallas import tpu as pltpu
from jax.experimental.pallas import tpu_sc as plsc

VOCAB = 21
EMB_DIM = 128
BATCH = 4096
SEQ_LEN = 50
_ROWS = BATCH * SEQ_LEN

_info = plsc.get_sparse_core_info()
_NC = _info.num_cores
_NS = _info.num_subcores
_NW = _NC * _NS
_WROWS = _ROWS // _NW
_CHUNK = 128
_NCHUNK = _WROWS // _CHUNK


def _tr_body(idx_ref, out_ref):
    out_ref[...] = jnp.transpose(idx_ref[...], (1, 0))


_transpose = pl.pallas_call(
    _tr_body,
    out_shape=jax.ShapeDtypeStruct((SEQ_LEN, BATCH), jnp.int32),
)


def _emb_body(idxT_hbm, table_hbm, out_hbm,
              idxT_v, rows0_v, rows1_v, table_sh,
              sem_i, sem_r0, sem_r1, sem_s0, sem_s1):
    sid = lax.axis_index("s")
    wid = sid * _NC + lax.axis_index("c")
    r0 = wid * _WROWS

    @pl.when(sid == 0)
    def _():
        pltpu.sync_copy(table_hbm, table_sh)

    pltpu.sync_copy(idxT_hbm.at[wid], idxT_v)
    plsc.subcore_barrier()

    def gather(c, rows, sem):
        pltpu.async_copy(table_sh.at[idxT_v.at[c]], rows, sem)

    def gather_wait(c, rows, sem):
        pltpu.make_async_copy(table_sh.at[idxT_v.at[c]], rows, sem).wait()

    def scatter(c, rows, sem):
        pltpu.async_copy(rows, out_hbm.at[pl.ds(r0 + c * _CHUNK, _CHUNK), :],
                         sem)

    def scatter_wait(c, rows, sem):
        pltpu.make_async_copy(rows,
                              out_hbm.at[pl.ds(r0 + c * _CHUNK, _CHUNK), :],
                              sem).wait()

    gather(0, rows0_v, sem_r0)
    gather(1, rows1_v, sem_r1)

    def pair(p, carry):
        c0 = 2 * p
        gather_wait(c0, rows0_v, sem_r0)
        scatter(c0, rows0_v, sem_s0)
        gather_wait(c0 + 1, rows1_v, sem_r1)
        scatter(c0 + 1, rows1_v, sem_s1)
        scatter_wait(c0, rows0_v, sem_s0)

        @pl.when(p < _NCHUNK // 2 - 1)
        def _():
            gather(c0 + 2, rows0_v, sem_r0)

        scatter_wait(c0 + 1, rows1_v, sem_s1)

        @pl.when(p < _NCHUNK // 2 - 1)
        def _():
            gather(c0 + 3, rows1_v, sem_r1)

        return carry

    lax.fori_loop(0, _NCHUNK // 2, pair, 0)


_emb_kernel = functools.partial(
    pl.kernel,
    mesh=plsc.VectorSubcoreMesh(core_axis_name="c", subcore_axis_name="s"),
    out_type=jax.ShapeDtypeStruct((_ROWS, EMB_DIM), jnp.float32),
    scratch_types=[
        pltpu.VMEM((_NCHUNK, _CHUNK), jnp.int32),
        pltpu.VMEM((_CHUNK, EMB_DIM), jnp.float32),
        pltpu.VMEM((_CHUNK, EMB_DIM), jnp.float32),
        pltpu.VMEM_SHARED((VOCAB, EMB_DIM), jnp.float32),
        pltpu.SemaphoreType.DMA,
        pltpu.SemaphoreType.DMA,
        pltpu.SemaphoreType.DMA,
        pltpu.SemaphoreType.DMA,
        pltpu.SemaphoreType.DMA,
    ],
)(_emb_body)


def kernel(indices, table):
    idx_t = _transpose(indices.astype(jnp.int32))
    idx_t = idx_t.reshape(_NW, _NCHUNK, _CHUNK)
    out = _emb_kernel(idx_t, table)
    return out.reshape(SEQ_LEN, BATCH, EMB_DIM)


# speedup vs baseline: 8.6211x; 1.3199x over previous
import functools

import jax
import jax.numpy as jnp
from jax import lax
from jax.experimental import pallas as pl
from jax.experimental.pallas import tpu as pltpu
from jax.experimental.pallas import tpu_sc as plsc

VOCAB = 21
EMB_DIM = 128
BATCH = 4096
SEQ_LEN = 50
_ROWS = BATCH * SEQ_LEN

_info = plsc.get_sparse_core_info()
_NC = _info.num_cores
_NS = _info.num_subcores
_NW = _NC * _NS
_WROWS = _ROWS // _NW
_CHUNK = 128
_NCHUNK = _WROWS // _CHUNK


def _tr_body(idx_ref, out_ref):
    out_ref[...] = jnp.transpose(idx_ref[...], (1, 0))


_transpose = pl.pallas_call(
    _tr_body,
    out_shape=jax.ShapeDtypeStruct((SEQ_LEN, BATCH), jnp.int32),
)


def _emb_body(idxT_hbm, table_hbm, out_hbm,
              idxT_v, rows0_v, rows1_v, rows2_v, rows3_v, table_sh,
              sem_r0, sem_r1, sem_r2, sem_r3,
              sem_s0, sem_s1, sem_s2, sem_s3):
    sid = lax.axis_index("s")
    wid = sid * _NC + lax.axis_index("c")
    r0 = wid * _WROWS

    @pl.when(sid == 0)
    def _():
        pltpu.sync_copy(table_hbm, table_sh)

    pltpu.sync_copy(idxT_hbm.at[wid], idxT_v)
    plsc.subcore_barrier()

    def gather(c, rows, sem):
        pltpu.async_copy(table_sh.at[idxT_v.at[c]], rows, sem)

    def gather_wait(c, rows, sem):
        pltpu.make_async_copy(table_sh.at[idxT_v.at[c]], rows, sem).wait()

    def scatter(c, rows, sem):
        pltpu.async_copy(rows, out_hbm.at[pl.ds(r0 + c * _CHUNK, _CHUNK), :],
                         sem)

    def scatter_wait(c, rows, sem):
        pltpu.make_async_copy(rows,
                              out_hbm.at[pl.ds(r0 + c * _CHUNK, _CHUNK), :],
                              sem).wait()

    rows = (rows0_v, rows1_v, rows2_v, rows3_v)
    sem_r = (sem_r0, sem_r1, sem_r2, sem_r3)
    sem_s = (sem_s0, sem_s1, sem_s2, sem_s3)
    _NBUF = 4

    for q in range(_NBUF):
        gather(q, rows[q], sem_r[q])

    def quad(p, carry):
        c0 = _NBUF * p
        for q in range(_NBUF):
            gather_wait(c0 + q, rows[q], sem_r[q])
            scatter(c0 + q, rows[q], sem_s[q])
        for q in range(_NBUF):
            scatter_wait(c0 + q, rows[q], sem_s[q])

            @pl.when(c0 + q + _NBUF < _NCHUNK)
            def _():
                gather(c0 + q + _NBUF, rows[q], sem_r[q])

        return carry

    lax.fori_loop(0, _NCHUNK // _NBUF, quad, 0)

    _REM = _NCHUNK % _NBUF
    _BASE = _NCHUNK - _REM
    for q in range(_REM):
        gather_wait(_BASE + q, rows[q], sem_r[q])
        scatter(_BASE + q, rows[q], sem_s[q])
    for q in range(_REM):
        scatter_wait(_BASE + q, rows[q], sem_s[q])


_emb_kernel = functools.partial(
    pl.kernel,
    mesh=plsc.VectorSubcoreMesh(core_axis_name="c", subcore_axis_name="s"),
    out_type=jax.ShapeDtypeStruct((_ROWS, EMB_DIM), jnp.float32),
    scratch_types=[
        pltpu.VMEM((_NCHUNK, _CHUNK), jnp.int32),
        pltpu.VMEM((_CHUNK, EMB_DIM), jnp.float32),
        pltpu.VMEM((_CHUNK, EMB_DIM), jnp.float32),
        pltpu.VMEM((_CHUNK, EMB_DIM), jnp.float32),
        pltpu.VMEM((_CHUNK, EMB_DIM), jnp.float32),
        pltpu.VMEM_SHARED((VOCAB, EMB_DIM), jnp.float32),
        pltpu.SemaphoreType.DMA,
        pltpu.SemaphoreType.DMA,
        pltpu.SemaphoreType.DMA,
        pltpu.SemaphoreType.DMA,
        pltpu.SemaphoreType.DMA,
        pltpu.SemaphoreType.DMA,
        pltpu.SemaphoreType.DMA,
        pltpu.SemaphoreType.DMA,
    ],
)(_emb_body)


def kernel(indices, table):
    idx_t = _transpose(indices.astype(jnp.int32))
    idx_t = idx_t.reshape(_NW, _NCHUNK, _CHUNK)
    out = _emb_kernel(idx_t, table)
    return out.reshape(SEQ_LEN, BATCH, EMB_DIM)


# speedup vs baseline: 8.6423x; 1.0025x over previous
import functools

import jax
import jax.numpy as jnp
from jax import lax
from jax.experimental import pallas as pl
from jax.experimental.pallas import tpu as pltpu
from jax.experimental.pallas import tpu_sc as plsc

VOCAB = 21
EMB_DIM = 128
BATCH = 4096
SEQ_LEN = 50
_ROWS = BATCH * SEQ_LEN

_info = plsc.get_sparse_core_info()
_NC = _info.num_cores
_NS = _info.num_subcores
_NW = _NC * _NS
_WROWS = _ROWS // _NW
_CHUNK = 128
_NCHUNK = _WROWS // _CHUNK


def _tr_body(idx_ref, out_ref):
    out_ref[...] = jnp.transpose(idx_ref[...], (1, 0))


_transpose = pl.pallas_call(
    _tr_body,
    out_shape=jax.ShapeDtypeStruct((SEQ_LEN, BATCH), jnp.int32),
)


def _emb_body(idxT_hbm, table_hbm, out_hbm,
              idxT_v, rows0_v, rows1_v, rows2_v, rows3_v, rows4_v, table_sh,
              sem_r0, sem_r1, sem_r2, sem_r3, sem_r4,
              sem_s0, sem_s1, sem_s2, sem_s3, sem_s4):
    sid = lax.axis_index("s")
    wid = sid * _NC + lax.axis_index("c")
    r0 = wid * _WROWS

    @pl.when(sid == 0)
    def _():
        pltpu.sync_copy(table_hbm, table_sh)

    pltpu.sync_copy(idxT_hbm.at[wid], idxT_v)
    plsc.subcore_barrier()

    def gather(c, rows, sem):
        pltpu.async_copy(table_sh.at[idxT_v.at[c]], rows, sem)

    def gather_wait(c, rows, sem):
        pltpu.make_async_copy(table_sh.at[idxT_v.at[c]], rows, sem).wait()

    def scatter(c, rows, sem):
        pltpu.async_copy(rows, out_hbm.at[pl.ds(r0 + c * _CHUNK, _CHUNK), :],
                         sem)

    def scatter_wait(c, rows, sem):
        pltpu.make_async_copy(rows,
                              out_hbm.at[pl.ds(r0 + c * _CHUNK, _CHUNK), :],
                              sem).wait()

    rows = (rows0_v, rows1_v, rows2_v, rows3_v, rows4_v)
    sem_r = (sem_r0, sem_r1, sem_r2, sem_r3, sem_r4)
    sem_s = (sem_s0, sem_s1, sem_s2, sem_s3, sem_s4)
    _NBUF = 5

    for q in range(_NBUF):
        gather(q, rows[q], sem_r[q])

    def quad(p, carry):
        c0 = _NBUF * p
        for q in range(_NBUF):
            gather_wait(c0 + q, rows[q], sem_r[q])
            scatter(c0 + q, rows[q], sem_s[q])
        for q in range(_NBUF):
            scatter_wait(c0 + q, rows[q], sem_s[q])

            @pl.when(c0 + q + _NBUF < _NCHUNK)
            def _():
                gather(c0 + q + _NBUF, rows[q], sem_r[q])

        return carry

    lax.fori_loop(0, _NCHUNK // _NBUF, quad, 0)

    _REM = _NCHUNK % _NBUF
    _BASE = _NCHUNK - _REM
    for q in range(_REM):
        gather_wait(_BASE + q, rows[q], sem_r[q])
        scatter(_BASE + q, rows[q], sem_s[q])
    for q in range(_REM):
        scatter_wait(_BASE + q, rows[q], sem_s[q])


_emb_kernel = functools.partial(
    pl.kernel,
    mesh=plsc.VectorSubcoreMesh(core_axis_name="c", subcore_axis_name="s"),
    out_type=jax.ShapeDtypeStruct((_ROWS, EMB_DIM), jnp.float32),
    scratch_types=[
        pltpu.VMEM((_NCHUNK, _CHUNK), jnp.int32),
        pltpu.VMEM((_CHUNK, EMB_DIM), jnp.float32),
        pltpu.VMEM((_CHUNK, EMB_DIM), jnp.float32),
        pltpu.VMEM((_CHUNK, EMB_DIM), jnp.float32),
        pltpu.VMEM((_CHUNK, EMB_DIM), jnp.float32),
        pltpu.VMEM((_CHUNK, EMB_DIM), jnp.float32),
        pltpu.VMEM_SHARED((VOCAB, EMB_DIM), jnp.float32),
        pltpu.SemaphoreType.DMA,
        pltpu.SemaphoreType.DMA,
        pltpu.SemaphoreType.DMA,
        pltpu.SemaphoreType.DMA,
        pltpu.SemaphoreType.DMA,
        pltpu.SemaphoreType.DMA,
        pltpu.SemaphoreType.DMA,
        pltpu.SemaphoreType.DMA,
        pltpu.SemaphoreType.DMA,
        pltpu.SemaphoreType.DMA,
    ],
)(_emb_body)


def kernel(indices, table):
    idx_t = _transpose(indices.astype(jnp.int32))
    idx_t = idx_t.reshape(_NW, _NCHUNK, _CHUNK)
    out = _emb_kernel(idx_t, table)
    return out.reshape(SEQ_LEN, BATCH, EMB_DIM)
